# baseline (speedup 1.0000x reference)
import functools

import jax
import jax.numpy as jnp
from jax import lax
from jax.experimental import pallas as pl
from jax.experimental.pallas import tpu as pltpu
from jax.experimental.pallas import tpu_sc as plsc

_SPECIAL_OFFSET = 20
_D = 256
_NCTX = 16
_NV = _D // 16
_C = 128
_G = 16
_EPS = 1e-5


def _rsqrt16(x):
    i = lax.bitcast_convert_type(x, jnp.int32)
    i = jnp.int32(0x5F3759DF) - lax.shift_right_logical(i, 1)
    y = lax.bitcast_convert_type(i, jnp.float32)
    for _ in range(3):
        y = y * (1.5 - 0.5 * x * y * y)
    return y


def _mlp_add_row(out_v, t, cf_v, w_v, bias_v, nk, b0):
    cfv = cf_v[t, pl.ds(0, _NCTX)]
    acc = [bias_v[b0, pl.ds(j * 16, 16)] for j in range(_NV)]
    for k in range(nk):
        s = jnp.broadcast_to(cfv[k], (16,))
        for j in range(_NV):
            acc[j] = acc[j] + s * w_v[k, pl.ds(j * 16, 16)]
    tot = acc[0]
    for j in range(1, _NV):
        tot = tot + acc[j]
    mu = jnp.broadcast_to(jnp.sum(tot) * (1.0 / _D), (16,))
    xc = [a - mu for a in acc]
    sq = xc[0] * xc[0]
    for j in range(1, _NV):
        sq = sq + xc[j] * xc[j]
    var = jnp.broadcast_to(jnp.sum(sq) * (1.0 / _D), (16,))
    rs = _rsqrt16(var + _EPS)
    off = t * _D
    for j in range(_NV):
        sl = pl.ds(off + j * 16, 16)
        o = xc[j] * rs * bias_v[b0 + 1, pl.ds(j * 16, 16)]
        o = o + bias_v[b0 + 2, pl.ds(j * 16, 16)]
        out_v[sl] = out_v[sl] + jnp.maximum(o, 0.0)


def _make_sc_call(T):
    tok_per_w = T // 32
    nchunk = tok_per_w // _C
    mesh = plsc.VectorSubcoreMesh(core_axis_name="c", subcore_axis_name="s")

    @functools.partial(
        pl.kernel,
        mesh=mesh,
        out_type=jax.ShapeDtypeStruct((T * _D,), jnp.float32),
        scratch_types=[
            pltpu.VMEM((_C,), jnp.int32),
            pltpu.VMEM((_C,), jnp.int32),
            pltpu.VMEM((_C, _NCTX), jnp.float32),
            pltpu.VMEM((_C * _D,), jnp.float32),
            pltpu.VMEM((_C * _D,), jnp.float32),
            pltpu.VMEM((16 * _D,), jnp.float32),
            pltpu.VMEM((3, _D), jnp.float32),
            pltpu.VMEM((_NCTX, _D), jnp.float32),
            pltpu.VMEM((6, _D), jnp.float32),
            pltpu.SemaphoreType.DMA,
            pltpu.SemaphoreType.DMA,
        ],
        compiler_params=pltpu.CompilerParams(needs_layout_passes=False),
    )
    def sc_call(ids_hbm, cf_hbm, tabx_hbm, wcls_hbm, wctx_hbm, bias_hbm,
                out_hbm, ids_v, sid_v, cf_v, out_a, out_b, tab_v, wcls_v,
                wctx_v, bias_v, sem_a, sem_b):
        wid = lax.axis_index("s") * 2 + lax.axis_index("c")
        base = wid * tok_per_w
        pltpu.sync_copy(tabx_hbm, tab_v)
        pltpu.sync_copy(wcls_hbm, wcls_v)
        pltpu.sync_copy(wctx_hbm, wctx_v)
        pltpu.sync_copy(bias_hbm, bias_v)
        lanes = lax.iota(jnp.int32, 16)

        def drain_mlp(out_v, mask_b, g16, w_v, nk, b0):
            def cond(mv):
                return jnp.max(mv) > 0

            def body(mv):
                iv = plsc.all_reduce_ffs(mv > 0)
                i = iv[0] if getattr(iv, "ndim", 0) else iv
                _mlp_add_row(out_v, g16 + i, cf_v, w_v, bias_v, nk, b0)
                ib = jnp.broadcast_to(i, (16,))
                return jnp.where(lanes == ib, jnp.zeros((16,), jnp.int32), mv)

            lax.while_loop(cond, body, mask_b.astype(jnp.int32))

        def fill(cidx, out_v):
            row0 = pl.multiple_of(base + cidx * _C, _C)
            pltpu.sync_copy(ids_hbm.at[pl.ds(row0, _C)], ids_v)
            pltpu.sync_copy(cf_hbm.at[pl.ds(row0, _C)], cf_v)

            def sids(g, carry):
                g16 = pl.multiple_of(g * _G, _G)
                idg = ids_v[pl.ds(g16, _G)]
                sid_v[pl.ds(g16, _G)] = jnp.where(
                    idg >= _SPECIAL_OFFSET, idg - _SPECIAL_OFFSET,
                    jnp.full((16,), 8, jnp.int32))
                return carry

            lax.fori_loop(0, _C // _G, sids, 0)

            @plsc.parallel_loop(0, _C, 1, unroll=4)
            def _(t):
                tb = jnp.broadcast_to(t, (16,))
                sidb = plsc.load_gather(sid_v, [tb])
                srcb = sidb * _D + lanes
                off = pl.multiple_of(t * _D, _D)
                for j in range(_NV):
                    v = plsc.load_gather(tab_v, [srcb + (j * 16)])
                    out_v[pl.ds(off + j * 16, 16)] = v

            def grp(g, carry):
                g16 = pl.multiple_of(g * _G, _G)
                idg = ids_v[pl.ds(g16, _G)]
                drain_mlp(out_v, idg == _SPECIAL_OFFSET, g16, wcls_v, 3, 0)
                drain_mlp(out_v, idg == _SPECIAL_OFFSET + 1, g16,
                          wctx_v, _NCTX, 3)
                return carry

            lax.fori_loop(0, _C // _G, grp, 0)
            return row0 * _D

        def pair(p, carry):
            r_a = fill(2 * p, out_a)
            cp_a = pltpu.async_copy(out_a, out_hbm.at[pl.ds(r_a, _C * _D)],
                                    sem_a)
            r_b = fill(2 * p + 1, out_b)
            cp_a.wait()
            pltpu.async_copy(out_b, out_hbm.at[pl.ds(r_b, _C * _D)],
                             sem_b).wait()
            return carry

        lax.fori_loop(0, nchunk // 2, pair, 0)

    return sc_call


def kernel(token_ids, context_features, special_table, W_cls, b_cls, g_cls,
           beta_cls, W_ctx, b_ctx, g_ctx, beta_ctx):
    B, S = token_ids.shape
    T = B * S
    assert T % (32 * 2 * _C) == 0
    ids = token_ids.astype(jnp.int32).reshape(T)
    cf = context_features.reshape(T, _NCTX)
    tab_ext = jnp.zeros((16, _D), jnp.float32).at[:8].set(
        special_table).reshape(-1)
    bias6 = jnp.stack([b_cls, g_cls, beta_cls, b_ctx, g_ctx, beta_ctx])
    out = _make_sc_call(T)(ids, cf, tab_ext, W_cls, W_ctx, bias6)
    return out.reshape(B, S, _D)


# speedup vs baseline: 1.0041x; 1.0041x over previous
import functools

import jax
import jax.numpy as jnp
from jax import lax
from jax.experimental import pallas as pl
from jax.experimental.pallas import tpu as pltpu
from jax.experimental.pallas import tpu_sc as plsc

_SPECIAL_OFFSET = 20
_D = 256
_NCTX = 16
_NV = _D // 16
_C = 128
_G = 16
_EPS = 1e-5


def _rsqrt16(x):
    i = lax.bitcast_convert_type(x, jnp.int32)
    i = jnp.int32(0x5F3759DF) - lax.shift_right_logical(i, 1)
    y = lax.bitcast_convert_type(i, jnp.float32)
    for _ in range(3):
        y = y * (1.5 - 0.5 * x * y * y)
    return y


def _mlp_add_row(out_v, t, cf_v, w_v, bias_v, nk, b0):
    cfv = cf_v[t, pl.ds(0, _NCTX)]
    acc = [bias_v[b0, pl.ds(j * 16, 16)] for j in range(_NV)]
    for k in range(nk):
        s = jnp.broadcast_to(cfv[k], (16,))
        for j in range(_NV):
            acc[j] = acc[j] + s * w_v[k, pl.ds(j * 16, 16)]
    tot = acc[0]
    for j in range(1, _NV):
        tot = tot + acc[j]
    mu = jnp.broadcast_to(jnp.sum(tot) * (1.0 / _D), (16,))
    xc = [a - mu for a in acc]
    sq = xc[0] * xc[0]
    for j in range(1, _NV):
        sq = sq + xc[j] * xc[j]
    var = jnp.broadcast_to(jnp.sum(sq) * (1.0 / _D), (16,))
    rs = _rsqrt16(var + _EPS)
    off = t * _D
    for j in range(_NV):
        sl = pl.ds(off + j * 16, 16)
        o = xc[j] * rs * bias_v[b0 + 1, pl.ds(j * 16, 16)]
        o = o + bias_v[b0 + 2, pl.ds(j * 16, 16)]
        out_v[sl] = out_v[sl] + jnp.maximum(o, 0.0)


def _make_sc_call(T):
    tok_per_w = T // 32
    nchunk = tok_per_w // _C
    mesh = plsc.VectorSubcoreMesh(core_axis_name="c", subcore_axis_name="s")

    @functools.partial(
        pl.kernel,
        mesh=mesh,
        out_type=jax.ShapeDtypeStruct((T * _D,), jnp.float32),
        scratch_types=[
            pltpu.VMEM((_C,), jnp.int32),
            pltpu.VMEM((_C,), jnp.int32),
            pltpu.VMEM((_C, _NCTX), jnp.float32),
            pltpu.VMEM((_C * _D,), jnp.float32),
            pltpu.VMEM((_C * _D,), jnp.float32),
            pltpu.VMEM((16 * _D,), jnp.float32),
            pltpu.VMEM((3, _D), jnp.float32),
            pltpu.VMEM((_NCTX, _D), jnp.float32),
            pltpu.VMEM((6, _D), jnp.float32),
            pltpu.SemaphoreType.DMA,
            pltpu.SemaphoreType.DMA,
        ],
        compiler_params=pltpu.CompilerParams(needs_layout_passes=False),
    )
    def sc_call(ids_hbm, cf_hbm, tabx_hbm, wcls_hbm, wctx_hbm, bias_hbm,
                out_hbm, ids_v, sid_v, cf_v, out_a, out_b, tab_v, wcls_v,
                wctx_v, bias_v, sem_a, sem_b):
        wid = lax.axis_index("s") * 2 + lax.axis_index("c")
        base = wid * tok_per_w
        pltpu.sync_copy(tabx_hbm, tab_v)
        pltpu.sync_copy(wcls_hbm, wcls_v)
        pltpu.sync_copy(wctx_hbm, wctx_v)
        pltpu.sync_copy(bias_hbm, bias_v)
        lanes = lax.iota(jnp.int32, 16)

        def drain_mlp(out_v, mask_b, g16, w_v, nk, b0):
            def cond(mv):
                return jnp.max(mv) > 0

            def body(mv):
                iv = plsc.all_reduce_ffs(mv > 0)
                i = iv[0] if getattr(iv, "ndim", 0) else iv
                _mlp_add_row(out_v, g16 + i, cf_v, w_v, bias_v, nk, b0)
                ib = jnp.broadcast_to(i, (16,))
                return jnp.where(lanes == ib, jnp.zeros((16,), jnp.int32), mv)

            lax.while_loop(cond, body, mask_b.astype(jnp.int32))

        def fill(cidx, out_v):
            row0 = pl.multiple_of(base + cidx * _C, _C)
            pltpu.sync_copy(ids_hbm.at[pl.ds(row0, _C)], ids_v)
            pltpu.sync_copy(cf_hbm.at[pl.ds(row0, _C)], cf_v)

            def sids(g, carry):
                g16 = pl.multiple_of(g * _G, _G)
                idg = ids_v[pl.ds(g16, _G)]
                sid_v[pl.ds(g16, _G)] = jnp.where(
                    idg >= _SPECIAL_OFFSET, idg - _SPECIAL_OFFSET,
                    jnp.full((16,), 8, jnp.int32))
                return carry

            lax.fori_loop(0, _C // _G, sids, 0)

            @plsc.parallel_loop(0, _C, 1, unroll=2)
            def _(t):
                tb = jnp.broadcast_to(t, (16,))
                sidb = plsc.load_gather(sid_v, [tb])
                srcb = sidb * _D + lanes
                off = pl.multiple_of(t * _D, _D)
                for j in range(_NV):
                    v = plsc.load_gather(tab_v, [srcb + (j * 16)])
                    out_v[pl.ds(off + j * 16, 16)] = v

            def grp(g, carry):
                g16 = pl.multiple_of(g * _G, _G)
                idg = ids_v[pl.ds(g16, _G)]
                drain_mlp(out_v, idg == _SPECIAL_OFFSET, g16, wcls_v, 3, 0)
                drain_mlp(out_v, idg == _SPECIAL_OFFSET + 1, g16,
                          wctx_v, _NCTX, 3)
                return carry

            lax.fori_loop(0, _C // _G, grp, 0)
            return row0 * _D

        def pair(p, carry):
            r_a = fill(2 * p, out_a)
            cp_a = pltpu.async_copy(out_a, out_hbm.at[pl.ds(r_a, _C * _D)],
                                    sem_a)
            r_b = fill(2 * p + 1, out_b)
            cp_a.wait()
            pltpu.async_copy(out_b, out_hbm.at[pl.ds(r_b, _C * _D)],
                             sem_b).wait()
            return carry

        lax.fori_loop(0, nchunk // 2, pair, 0)

    return sc_call


def kernel(token_ids, context_features, special_table, W_cls, b_cls, g_cls,
           beta_cls, W_ctx, b_ctx, g_ctx, beta_ctx):
    B, S = token_ids.shape
    T = B * S
    assert T % (32 * 2 * _C) == 0
    ids = token_ids.astype(jnp.int32).reshape(T)
    cf = context_features.reshape(T, _NCTX)
    tab_ext = jnp.zeros((16, _D), jnp.float32).at[:8].set(
        special_table).reshape(-1)
    bias6 = jnp.stack([b_cls, g_cls, beta_cls, b_ctx, g_ctx, beta_ctx])
    out = _make_sc_call(T)(ids, cf, tab_ext, W_cls, W_ctx, bias6)
    return out.reshape(B, S, _D)
